# Initial kernel scaffold; baseline (speedup 1.0000x reference)
import jax
import jax.numpy as jnp
from jax.experimental import pallas as pl


def kernel(coordinates, features, W1, W2):
    raise NotImplementedError("write your pallas kernel here")


# speedup vs baseline: 9.1639x; 9.1639x over previous
import functools

import jax
import jax.numpy as jnp
from jax import lax
from jax.experimental import pallas as pl
from jax.experimental.pallas import tpu as pltpu
from jax.experimental.pallas import tpu_sc as plsc

N = 4096
K = 16
C = 64
BN = 256
NBLK = N // BN
TOT = float(N * K)
NEG = -3.0e38


def _knn_body(ct_ref, call_ref, ft_ref, w1ct_ref, w1ft_ref,
              idx_ref, ut_ref, b1t_ref):
    xbT = ct_ref[...]
    xall = call_ref[...]
    ip2 = 2.0 * jnp.dot(xbT, xall, preferred_element_type=jnp.float32)
    xxb = jnp.sum(xbT * xbT, axis=1, keepdims=True)
    xxa = jnp.sum(xall * xall, axis=0, keepdims=True)
    d = ip2 - xxb - xxa
    iota = lax.broadcasted_iota(jnp.int32, d.shape, 1)
    cols = []
    for _ in range(K):
        m = jnp.max(d, axis=1, keepdims=True)
        ismax = d == m
        col = jnp.min(jnp.where(ismax, iota, N), axis=1, keepdims=True)
        cols.append(col)
        d = jnp.where(iota == col, NEG, d)
    idx_ref[...] = jnp.concatenate(cols, axis=1)
    ut = jnp.dot(xbT, w1ct_ref[...], preferred_element_type=jnp.float32)
    ut_ref[...] = jnp.concatenate([ut, jnp.zeros_like(ut)], axis=1)
    b1t_ref[...] = (
        jnp.dot(ft_ref[...], w1ft_ref[...], preferred_element_type=jnp.float32)
        - ut)


def _knn_call(xT, x, fT, w1cT, w1fT):
    return pl.pallas_call(
        _knn_body,
        grid=(NBLK,),
        in_specs=[
            pl.BlockSpec((BN, 3), lambda i: (i, 0)),
            pl.BlockSpec((3, N), lambda i: (0, 0)),
            pl.BlockSpec((BN, C), lambda i: (i, 0)),
            pl.BlockSpec((3, C), lambda i: (0, 0)),
            pl.BlockSpec((C, C), lambda i: (0, 0)),
        ],
        out_specs=[
            pl.BlockSpec((BN, K), lambda i: (i, 0)),
            pl.BlockSpec((BN, 2 * C), lambda i: (i, 0)),
            pl.BlockSpec((BN, C), lambda i: (i, 0)),
        ],
        out_shape=[
            jax.ShapeDtypeStruct((N, K), jnp.int32),
            jax.ShapeDtypeStruct((N, 2 * C), jnp.float32),
            jax.ShapeDtypeStruct((N, C), jnp.float32),
        ],
    )(xT, x, fT, w1cT, w1fT)


def _sc_gather(table, idx2d):
    mesh = plsc.VectorSubcoreMesh(core_axis_name="c", subcore_axis_name="s")

    @functools.partial(
        pl.kernel,
        mesh=mesh,
        out_type=jax.ShapeDtypeStruct((N * K, 2 * C), jnp.float32),
        scratch_types=[
            pltpu.VMEM((16, 128), jnp.int32),
            pltpu.VMEM((512, 2 * C), jnp.float32),
            pltpu.SemaphoreType.DMA,
        ],
    )
    def k(table_hbm, idx_hbm, out_hbm, idx_v, buf_v, sem):
        wid = lax.axis_index("s") * 2 + lax.axis_index("c")
        pltpu.sync_copy(idx_hbm.at[pl.ds(wid * 16, 16)], idx_v)
        for chunk in range(4):
            handles = []
            for j in range(4):
                r = chunk * 4 + j
                handles.append(pltpu.async_copy(
                    table_hbm.at[idx_v.at[r]],
                    buf_v.at[pl.ds(j * 128, 128)], sem))
            for h in handles:
                h.wait()
            pltpu.sync_copy(
                buf_v, out_hbm.at[pl.ds(wid * 2048 + chunk * 512, 512)])

    return k(table, idx2d)


def _stats_body(g_ref, bt_ref, stats_ref):
    @pl.when(pl.program_id(0) == 0)
    def _():
        stats_ref[...] = jnp.zeros_like(stats_ref)
    y = g_ref[:, :, 0:C] + bt_ref[...][:, None, :]
    s = jnp.sum(jnp.sum(y, axis=1), axis=0, keepdims=True)
    sq = jnp.sum(jnp.sum(y * y, axis=1), axis=0, keepdims=True)
    stats_ref[...] += jnp.concatenate([s, sq], axis=0)


def _stats_call(g3, bT):
    return pl.pallas_call(
        _stats_body,
        grid=(NBLK,),
        in_specs=[
            pl.BlockSpec((BN, K, 2 * C), lambda i: (i, 0, 0)),
            pl.BlockSpec((BN, C), lambda i: (i, 0)),
        ],
        out_specs=pl.BlockSpec((2, C), lambda i: (0, 0)),
        out_shape=jax.ShapeDtypeStruct((2, C), jnp.float32),
    )(g3, bT)


def _normalize(g_ref, bt_ref, stats_ref):
    s = stats_ref[0:1, :]
    sq = stats_ref[1:2, :]
    mu = s / TOT
    var = sq / TOT - mu * mu
    inv = lax.rsqrt(var + 1e-5)
    y = g_ref[:, :, 0:C] + bt_ref[...][:, None, :]
    yn = (y - mu[None]) * inv[None]
    act = jnp.where(yn > 0, yn, 0.2 * yn)
    return jnp.mean(act, axis=1)


def _final1_body(g_ref, bt_ref, stats_ref, at_ref, bmt_ref, zt_ref, c2t_ref):
    x1 = _normalize(g_ref, bt_ref, stats_ref)
    z = jnp.dot(x1, at_ref[...], preferred_element_type=jnp.float32)
    zt_ref[...] = jnp.concatenate([z, jnp.zeros_like(z)], axis=1)
    c2t_ref[...] = jnp.dot(x1, bmt_ref[...], preferred_element_type=jnp.float32)


def _final1_call(g3, bT, stats, aT, bmT):
    return pl.pallas_call(
        _final1_body,
        grid=(NBLK,),
        in_specs=[
            pl.BlockSpec((BN, K, 2 * C), lambda i: (i, 0, 0)),
            pl.BlockSpec((BN, C), lambda i: (i, 0)),
            pl.BlockSpec((2, C), lambda i: (0, 0)),
            pl.BlockSpec((C, C), lambda i: (0, 0)),
            pl.BlockSpec((C, C), lambda i: (0, 0)),
        ],
        out_specs=[
            pl.BlockSpec((BN, 2 * C), lambda i: (i, 0)),
            pl.BlockSpec((BN, C), lambda i: (i, 0)),
        ],
        out_shape=[
            jax.ShapeDtypeStruct((N, 2 * C), jnp.float32),
            jax.ShapeDtypeStruct((N, C), jnp.float32),
        ],
    )(g3, bT, stats, aT, bmT)


def _final2_body(g_ref, bt_ref, stats_ref, out_ref):
    out_ref[...] = _normalize(g_ref, bt_ref, stats_ref)


def _final2_call(g3, bT, stats):
    return pl.pallas_call(
        _final2_body,
        grid=(NBLK,),
        in_specs=[
            pl.BlockSpec((BN, K, 2 * C), lambda i: (i, 0, 0)),
            pl.BlockSpec((BN, C), lambda i: (i, 0)),
            pl.BlockSpec((2, C), lambda i: (0, 0)),
        ],
        out_specs=pl.BlockSpec((BN, C), lambda i: (i, 0)),
        out_shape=jax.ShapeDtypeStruct((N, C), jnp.float32),
    )(g3, bT, stats)


def kernel(coordinates, features, W1, W2):
    x = coordinates[0]
    xT = jnp.transpose(x)
    fT = jnp.transpose(features[0])
    w1cT = jnp.transpose(W1[:, C:])
    w1fT = jnp.transpose(W1[:, :C])
    aT = jnp.transpose(W2[:, :C])
    bmT = jnp.transpose(W2[:, C:] - W2[:, :C])

    idx, uT, b1T = _knn_call(xT, x, fT, w1cT, w1fT)
    idx2d = idx.reshape(N * K // 128, 128)

    g1 = _sc_gather(uT, idx2d).reshape(N, K, 2 * C)
    stats1 = _stats_call(g1, b1T)
    zT, c2T = _final1_call(g1, b1T, stats1, aT, bmT)

    g2 = _sc_gather(zT, idx2d).reshape(N, K, 2 * C)
    stats2 = _stats_call(g2, c2T)
    x2T = _final2_call(g2, c2T, stats2)

    return jnp.transpose(x2T)[None]


# speedup vs baseline: 11.4583x; 1.2504x over previous
import functools

import jax
import jax.numpy as jnp
from jax import lax
from jax.experimental import pallas as pl
from jax.experimental.pallas import tpu as pltpu
from jax.experimental.pallas import tpu_sc as plsc

N = 4096
K = 16
C = 64
BN = 256
NBLK = N // BN
TOT = float(N * K)
NEG = -3.0e38


def _knn_body(ct_ref, call_ref, ft_ref, w1ct_ref, w1ft_ref,
              idx_ref, ut_ref, b1t_ref):
    xbT = ct_ref[...]
    xall = call_ref[...]
    ip2 = 2.0 * jnp.dot(xbT, xall, preferred_element_type=jnp.float32)
    xxb = jnp.sum(xbT * xbT, axis=1, keepdims=True)
    xxa = jnp.sum(xall * xall, axis=0, keepdims=True)
    d = ip2 - xxb - xxa
    iota = lax.broadcasted_iota(jnp.int32, d.shape, 1)
    s = lax.bitcast_convert_type(d, jnp.int32)
    t = jnp.where(s < 0, s ^ 0x7FFFFFFF, s)
    key = (t & -4096) | ((N - 1) - iota)
    mprev = jnp.full((d.shape[0], 1), 0x7FFFFFFF, jnp.int32)
    ms = []
    for _ in range(K):
        masked = jnp.where(key < mprev, key, jnp.int32(-0x80000000))
        mprev = jnp.max(masked, axis=1, keepdims=True)
        ms.append(mprev)
    mcat = jnp.concatenate(ms, axis=1)
    idx_ref[...] = (N - 1) - (mcat & 0xFFF)
    ut = jnp.dot(xbT, w1ct_ref[...], preferred_element_type=jnp.float32)
    ut_ref[...] = jnp.concatenate([ut, jnp.zeros_like(ut)], axis=1)
    b1t_ref[...] = (
        jnp.dot(ft_ref[...], w1ft_ref[...], preferred_element_type=jnp.float32)
        - ut)


def _knn_call(xT, x, fT, w1cT, w1fT):
    return pl.pallas_call(
        _knn_body,
        grid=(NBLK,),
        in_specs=[
            pl.BlockSpec((BN, 3), lambda i: (i, 0)),
            pl.BlockSpec((3, N), lambda i: (0, 0)),
            pl.BlockSpec((BN, C), lambda i: (i, 0)),
            pl.BlockSpec((3, C), lambda i: (0, 0)),
            pl.BlockSpec((C, C), lambda i: (0, 0)),
        ],
        out_specs=[
            pl.BlockSpec((BN, K), lambda i: (i, 0)),
            pl.BlockSpec((BN, 2 * C), lambda i: (i, 0)),
            pl.BlockSpec((BN, C), lambda i: (i, 0)),
        ],
        out_shape=[
            jax.ShapeDtypeStruct((N, K), jnp.int32),
            jax.ShapeDtypeStruct((N, 2 * C), jnp.float32),
            jax.ShapeDtypeStruct((N, C), jnp.float32),
        ],
    )(xT, x, fT, w1cT, w1fT)


def _sc_gather(table, idx2d):
    mesh = plsc.VectorSubcoreMesh(core_axis_name="c", subcore_axis_name="s")

    @functools.partial(
        pl.kernel,
        mesh=mesh,
        out_type=jax.ShapeDtypeStruct((N * K, 2 * C), jnp.float32),
        scratch_types=[
            pltpu.VMEM((16, 128), jnp.int32),
            pltpu.VMEM((512, 2 * C), jnp.float32),
            pltpu.SemaphoreType.DMA,
        ],
    )
    def k(table_hbm, idx_hbm, out_hbm, idx_v, buf_v, sem):
        wid = lax.axis_index("s") * 2 + lax.axis_index("c")
        pltpu.sync_copy(idx_hbm.at[pl.ds(wid * 16, 16)], idx_v)
        for chunk in range(4):
            handles = []
            for j in range(4):
                r = chunk * 4 + j
                handles.append(pltpu.async_copy(
                    table_hbm.at[idx_v.at[r]],
                    buf_v.at[pl.ds(j * 128, 128)], sem))
            for h in handles:
                h.wait()
            pltpu.sync_copy(
                buf_v, out_hbm.at[pl.ds(wid * 2048 + chunk * 512, 512)])

    return k(table, idx2d)


def _bn_phase1(y, i, acc_ref):
    @pl.when(i == 0)
    def _():
        acc_ref[...] = jnp.zeros_like(acc_ref)

    @pl.when(i < NBLK)
    def _():
        s = jnp.sum(jnp.sum(y, axis=1), axis=0, keepdims=True)
        sq = jnp.sum(jnp.sum(y * y, axis=1), axis=0, keepdims=True)
        acc_ref[...] += jnp.concatenate([s, sq], axis=0)


def _bn_apply(y, acc_ref):
    mu = acc_ref[0:1, :] / TOT
    var = acc_ref[1:2, :] / TOT - mu * mu
    inv = lax.rsqrt(var + 1e-5)
    yn = (y - mu[None]) * inv[None]
    act = jnp.where(yn > 0, yn, 0.2 * yn)
    return jnp.mean(act, axis=1)


def _final1_body(g_ref, bt_ref, at_ref, bmt_ref, zt_ref, c2t_ref, acc_ref):
    i = pl.program_id(0)
    y = g_ref[:, :, 0:C] + bt_ref[...][:, None, :]
    _bn_phase1(y, i, acc_ref)

    @pl.when(i >= NBLK)
    def _():
        x1 = _bn_apply(y, acc_ref)
        z = jnp.dot(x1, at_ref[...], preferred_element_type=jnp.float32)
        zt_ref[...] = jnp.concatenate([z, jnp.zeros_like(z)], axis=1)
        c2t_ref[...] = jnp.dot(x1, bmt_ref[...],
                               preferred_element_type=jnp.float32)


def _final1_call(g3, bT, aT, bmT):
    return pl.pallas_call(
        _final1_body,
        grid=(2 * NBLK,),
        in_specs=[
            pl.BlockSpec((BN, K, 2 * C), lambda i: (i % NBLK, 0, 0)),
            pl.BlockSpec((BN, C), lambda i: (i % NBLK, 0)),
            pl.BlockSpec((C, C), lambda i: (0, 0)),
            pl.BlockSpec((C, C), lambda i: (0, 0)),
        ],
        out_specs=[
            pl.BlockSpec((BN, 2 * C), lambda i: (i % NBLK, 0)),
            pl.BlockSpec((BN, C), lambda i: (i % NBLK, 0)),
        ],
        out_shape=[
            jax.ShapeDtypeStruct((N, 2 * C), jnp.float32),
            jax.ShapeDtypeStruct((N, C), jnp.float32),
        ],
        scratch_shapes=[pltpu.VMEM((2, C), jnp.float32)],
    )(g3, bT, aT, bmT)


def _final2_body(g_ref, bt_ref, out_ref, acc_ref):
    i = pl.program_id(0)
    y = g_ref[:, :, 0:C] + bt_ref[...][:, None, :]
    _bn_phase1(y, i, acc_ref)

    @pl.when(i >= NBLK)
    def _():
        out_ref[...] = _bn_apply(y, acc_ref)


def _final2_call(g3, bT):
    return pl.pallas_call(
        _final2_body,
        grid=(2 * NBLK,),
        in_specs=[
            pl.BlockSpec((BN, K, 2 * C), lambda i: (i % NBLK, 0, 0)),
            pl.BlockSpec((BN, C), lambda i: (i % NBLK, 0)),
        ],
        out_specs=pl.BlockSpec((BN, C), lambda i: (i % NBLK, 0)),
        out_shape=jax.ShapeDtypeStruct((N, C), jnp.float32),
        scratch_shapes=[pltpu.VMEM((2, C), jnp.float32)],
    )(g3, bT)


def kernel(coordinates, features, W1, W2):
    x = coordinates[0]
    xT = jnp.transpose(x)
    fT = jnp.transpose(features[0])
    w1cT = jnp.transpose(W1[:, C:])
    w1fT = jnp.transpose(W1[:, :C])
    aT = jnp.transpose(W2[:, :C])
    bmT = jnp.transpose(W2[:, C:] - W2[:, :C])

    idx, uT, b1T = _knn_call(xT, x, fT, w1cT, w1fT)
    idx2d = idx.reshape(N * K // 128, 128)

    g1 = _sc_gather(uT, idx2d).reshape(N, K, 2 * C)
    zT, c2T = _final1_call(g1, b1T, aT, bmT)

    g2 = _sc_gather(zT, idx2d).reshape(N, K, 2 * C)
    x2T = _final2_call(g2, c2T)

    return jnp.transpose(x2T)[None]


# speedup vs baseline: 11.6947x; 1.0206x over previous
import functools

import jax
import jax.numpy as jnp
from jax import lax
from jax.experimental import pallas as pl
from jax.experimental.pallas import tpu as pltpu
from jax.experimental.pallas import tpu_sc as plsc

N = 4096
K = 16
C = 64
BN = 256
NBLK = N // BN
TOT = float(N * K)
NEG = -3.0e38


def _knn_body(xbt_ref, call_ref, fb_ref, w1c_ref, w1f_ref,
              idx_ref, ut_ref, b1t_ref):
    xbT = xbt_ref[...]
    xall = call_ref[...]
    ip2 = 2.0 * jnp.dot(xbT, xall, preferred_element_type=jnp.float32)
    xxb = jnp.sum(xbT * xbT, axis=1, keepdims=True)
    xxa = jnp.sum(xall * xall, axis=0, keepdims=True)
    d = ip2 - xxb - xxa
    iota = lax.broadcasted_iota(jnp.int32, d.shape, 1)
    s = lax.bitcast_convert_type(d, jnp.int32)
    t = jnp.where(s < 0, s ^ 0x7FFFFFFF, s)
    key = (t & -4096) | ((N - 1) - iota)
    mprev = jnp.full((d.shape[0], 1), 0x7FFFFFFF, jnp.int32)
    ms = []
    for _ in range(K):
        masked = jnp.where(key < mprev, key, jnp.int32(-0x80000000))
        mprev = jnp.max(masked, axis=1, keepdims=True)
        ms.append(mprev)
    mcat = jnp.concatenate(ms, axis=1)
    idx_ref[...] = (N - 1) - (mcat & 0xFFF)
    ut = lax.dot_general(xbT, w1c_ref[...], (((1,), (1,)), ((), ())),
                         preferred_element_type=jnp.float32)
    ut_ref[...] = jnp.concatenate([ut, jnp.zeros_like(ut)], axis=1)
    b1t_ref[...] = (
        lax.dot_general(fb_ref[...], w1f_ref[...], (((0,), (1,)), ((), ())),
                        preferred_element_type=jnp.float32)
        - ut)


def _knn_call(xT, x, f, w1c, w1f):
    return pl.pallas_call(
        _knn_body,
        grid=(NBLK,),
        in_specs=[
            pl.BlockSpec((BN, 3), lambda i: (i, 0)),
            pl.BlockSpec((3, N), lambda i: (0, 0)),
            pl.BlockSpec((C, BN), lambda i: (0, i)),
            pl.BlockSpec((C, 3), lambda i: (0, 0)),
            pl.BlockSpec((C, C), lambda i: (0, 0)),
        ],
        out_specs=[
            pl.BlockSpec((BN, K), lambda i: (i, 0)),
            pl.BlockSpec((BN, 2 * C), lambda i: (i, 0)),
            pl.BlockSpec((BN, C), lambda i: (i, 0)),
        ],
        out_shape=[
            jax.ShapeDtypeStruct((N, K), jnp.int32),
            jax.ShapeDtypeStruct((N, 2 * C), jnp.float32),
            jax.ShapeDtypeStruct((N, C), jnp.float32),
        ],
    )(xT, x, f, w1c, w1f)


def _sc_gather(table, idx2d):
    mesh = plsc.VectorSubcoreMesh(core_axis_name="c", subcore_axis_name="s")

    @functools.partial(
        pl.kernel,
        mesh=mesh,
        out_type=jax.ShapeDtypeStruct((N * K, 2 * C), jnp.float32),
        scratch_types=[
            pltpu.VMEM((16, 128), jnp.int32),
            pltpu.VMEM((512, 2 * C), jnp.float32),
            pltpu.SemaphoreType.DMA,
        ],
    )
    def k(table_hbm, idx_hbm, out_hbm, idx_v, buf_v, sem):
        wid = lax.axis_index("s") * 2 + lax.axis_index("c")
        pltpu.sync_copy(idx_hbm.at[pl.ds(wid * 16, 16)], idx_v)
        for chunk in range(4):
            handles = []
            for j in range(4):
                r = chunk * 4 + j
                handles.append(pltpu.async_copy(
                    table_hbm.at[idx_v.at[r]],
                    buf_v.at[pl.ds(j * 128, 128)], sem))
            for h in handles:
                h.wait()
            pltpu.sync_copy(
                buf_v, out_hbm.at[pl.ds(wid * 2048 + chunk * 512, 512)])

    return k(table, idx2d)


def _bn_phase1(y, i, acc_ref):
    @pl.when(i == 0)
    def _():
        acc_ref[...] = jnp.zeros_like(acc_ref)

    @pl.when(i < NBLK)
    def _():
        s = jnp.sum(jnp.sum(y, axis=1), axis=0, keepdims=True)
        sq = jnp.sum(jnp.sum(y * y, axis=1), axis=0, keepdims=True)
        acc_ref[...] += jnp.concatenate([s, sq], axis=0)


def _bn_apply(y, acc_ref):
    mu = acc_ref[0:1, :] / TOT
    var = acc_ref[1:2, :] / TOT - mu * mu
    inv = lax.rsqrt(var + 1e-5)
    yn = (y - mu[None]) * inv[None]
    act = jnp.where(yn > 0, yn, 0.2 * yn)
    return jnp.mean(act, axis=1)


def _final1_body(g_ref, bt_ref, at_ref, bmt_ref, zt_ref, c2t_ref, acc_ref):
    i = pl.program_id(0)
    y = g_ref[:, :, 0:C] + bt_ref[...][:, None, :]
    _bn_phase1(y, i, acc_ref)

    @pl.when(i >= NBLK)
    def _():
        x1 = _bn_apply(y, acc_ref)
        z = jnp.dot(x1, at_ref[...], preferred_element_type=jnp.float32)
        zt_ref[...] = jnp.concatenate([z, jnp.zeros_like(z)], axis=1)
        c2t_ref[...] = jnp.dot(x1, bmt_ref[...],
                               preferred_element_type=jnp.float32)


def _final1_call(g3, bT, aT, bmT):
    return pl.pallas_call(
        _final1_body,
        grid=(2 * NBLK,),
        in_specs=[
            pl.BlockSpec((BN, K, 2 * C), lambda i: (i % NBLK, 0, 0)),
            pl.BlockSpec((BN, C), lambda i: (i % NBLK, 0)),
            pl.BlockSpec((C, C), lambda i: (0, 0)),
            pl.BlockSpec((C, C), lambda i: (0, 0)),
        ],
        out_specs=[
            pl.BlockSpec((BN, 2 * C), lambda i: (i % NBLK, 0)),
            pl.BlockSpec((BN, C), lambda i: (i % NBLK, 0)),
        ],
        out_shape=[
            jax.ShapeDtypeStruct((N, 2 * C), jnp.float32),
            jax.ShapeDtypeStruct((N, C), jnp.float32),
        ],
        scratch_shapes=[pltpu.VMEM((2, C), jnp.float32)],
    )(g3, bT, aT, bmT)


def _final2_body(g_ref, bt_ref, out_ref, acc_ref):
    i = pl.program_id(0)
    y = g_ref[:, :, 0:C] + bt_ref[...][:, None, :]
    _bn_phase1(y, i, acc_ref)

    @pl.when(i >= NBLK)
    def _():
        out_ref[...] = jnp.transpose(_bn_apply(y, acc_ref))


def _final2_call(g3, bT):
    return pl.pallas_call(
        _final2_body,
        grid=(2 * NBLK,),
        in_specs=[
            pl.BlockSpec((BN, K, 2 * C), lambda i: (i % NBLK, 0, 0)),
            pl.BlockSpec((BN, C), lambda i: (i % NBLK, 0)),
        ],
        out_specs=pl.BlockSpec((C, BN), lambda i: (0, i % NBLK)),
        out_shape=jax.ShapeDtypeStruct((C, N), jnp.float32),
        scratch_shapes=[pltpu.VMEM((2, C), jnp.float32)],
    )(g3, bT)


def kernel(coordinates, features, W1, W2):
    x = coordinates[0]
    f = features[0]
    aT = jnp.transpose(W2[:, :C])
    bmT = jnp.transpose(W2[:, C:] - W2[:, :C])

    idx, uT, b1T = _knn_call(jnp.transpose(x), x, f, W1[:, C:], W1[:, :C])
    idx2d = idx.reshape(N * K // 128, 128)

    g1 = _sc_gather(uT, idx2d).reshape(N, K, 2 * C)
    zT, c2T = _final1_call(g1, b1T, aT, bmT)

    g2 = _sc_gather(zT, idx2d).reshape(N, K, 2 * C)
    x2 = _final2_call(g2, c2T)

    return x2[None]


# speedup vs baseline: 12.5879x; 1.0764x over previous
import functools

import jax
import jax.numpy as jnp
from jax import lax
from jax.experimental import pallas as pl
from jax.experimental.pallas import tpu as pltpu
from jax.experimental.pallas import tpu_sc as plsc

N = 4096
K = 16
C = 64
BN = 256
NBLK = N // BN
TOT = float(N * K)
NEG = -3.0e38


def _knn_body(xbt_ref, call_ref, fb_ref, w1c_ref, w1f_ref,
              idx_ref, ut_ref, b1t_ref):
    xbT = xbt_ref[...]
    xall = call_ref[...]
    ip2 = 2.0 * jnp.dot(xbT, xall, preferred_element_type=jnp.float32)
    xxb = jnp.sum(xbT * xbT, axis=1, keepdims=True)
    xxa = jnp.sum(xall * xall, axis=0, keepdims=True)
    d = ip2 - xxb - xxa
    iota = lax.broadcasted_iota(jnp.int32, d.shape, 1)
    s = lax.bitcast_convert_type(d, jnp.int32)
    t = jnp.where(s < 0, s ^ 0x7FFFFFFF, s)
    key = (t & -4096) | ((N - 1) - iota)
    mprev = jnp.full((d.shape[0], 1), 0x7FFFFFFF, jnp.int32)
    ms = []
    for _ in range(K):
        masked = jnp.where(key < mprev, key, jnp.int32(-0x80000000))
        mprev = jnp.max(masked, axis=1, keepdims=True)
        ms.append(mprev)
    mcat = jnp.concatenate(ms, axis=1)
    idx_ref[...] = (N - 1) - (mcat & 0xFFF)
    ut = lax.dot_general(xbT, w1c_ref[...], (((1,), (1,)), ((), ())),
                         preferred_element_type=jnp.float32)
    ut_ref[...] = jnp.concatenate([ut, jnp.zeros_like(ut)], axis=1)
    b1 = (
        lax.dot_general(fb_ref[...], w1f_ref[...], (((0,), (1,)), ((), ())),
                        preferred_element_type=jnp.float32)
        - ut)
    b1t_ref[...] = jnp.concatenate([b1, jnp.zeros_like(b1)], axis=1)


def _knn_call(xT, x, f, w1c, w1f):
    return pl.pallas_call(
        _knn_body,
        grid=(NBLK,),
        in_specs=[
            pl.BlockSpec((BN, 3), lambda i: (i, 0)),
            pl.BlockSpec((3, N), lambda i: (0, 0)),
            pl.BlockSpec((C, BN), lambda i: (0, i)),
            pl.BlockSpec((C, 3), lambda i: (0, 0)),
            pl.BlockSpec((C, C), lambda i: (0, 0)),
        ],
        out_specs=[
            pl.BlockSpec((BN, K), lambda i: (i, 0)),
            pl.BlockSpec((BN, 2 * C), lambda i: (i, 0)),
            pl.BlockSpec((BN, 2 * C), lambda i: (i, 0)),
        ],
        out_shape=[
            jax.ShapeDtypeStruct((N, K), jnp.int32),
            jax.ShapeDtypeStruct((N, 2 * C), jnp.float32),
            jax.ShapeDtypeStruct((N, 2 * C), jnp.float32),
        ],
    )(xT, x, f, w1c, w1f)


def _sc_gather(table, idx2d):
    mesh = plsc.VectorSubcoreMesh(core_axis_name="c", subcore_axis_name="s")

    @functools.partial(
        pl.kernel,
        mesh=mesh,
        out_type=jax.ShapeDtypeStruct((N * K, 2 * C), jnp.float32),
        scratch_types=[
            pltpu.VMEM((16, 128), jnp.int32),
            pltpu.VMEM((512, 2 * C), jnp.float32),
            pltpu.SemaphoreType.DMA,
        ],
    )
    def k(table_hbm, idx_hbm, out_hbm, idx_v, buf_v, sem):
        wid = lax.axis_index("s") * 2 + lax.axis_index("c")
        pltpu.sync_copy(idx_hbm.at[pl.ds(wid * 16, 16)], idx_v)
        for chunk in range(4):
            handles = []
            for j in range(4):
                r = chunk * 4 + j
                handles.append(pltpu.async_copy(
                    table_hbm.at[idx_v.at[r]],
                    buf_v.at[pl.ds(j * 128, 128)], sem))
            for h in handles:
                h.wait()
            pltpu.sync_copy(
                buf_v, out_hbm.at[pl.ds(wid * 2048 + chunk * 512, 512)])

    return k(table, idx2d)


def _bn_step(g_ref, bt_ref, ybuf_ref, acc_ref, i):
    y = g_ref[...] + bt_ref[...][:, None, :]
    ybuf_ref[pl.ds(i * BN, BN)] = y

    @pl.when(i == 0)
    def _():
        acc_ref[...] = jnp.zeros_like(acc_ref)

    s = jnp.sum(jnp.sum(y, axis=1), axis=0, keepdims=True)
    sq = jnp.sum(jnp.sum(y * y, axis=1), axis=0, keepdims=True)
    acc_ref[...] += jnp.concatenate([s, sq], axis=0)


def _bn_inv(acc_ref):
    mu = acc_ref[0:1, :] / TOT
    var = acc_ref[1:2, :] / TOT - mu * mu
    return mu, lax.rsqrt(var + 1e-5)


def _bn_apply(yb, mu, inv):
    yn = (yb - mu[None]) * inv[None]
    act = jnp.where(yn > 0, yn, 0.2 * yn)
    return jnp.mean(act, axis=1)


def _final1_body(g_ref, bt_ref, at_ref, bmt_ref, zt_ref, c2t_ref,
                 ybuf_ref, acc_ref):
    i = pl.program_id(0)
    _bn_step(g_ref, bt_ref, ybuf_ref, acc_ref, i)

    @pl.when(i == NBLK - 1)
    def _():
        mu, inv = _bn_inv(acc_ref)

        def body(b, carry):
            x1 = _bn_apply(ybuf_ref[pl.ds(b * BN, BN)], mu, inv)[:, 0:C]
            z = jnp.dot(x1, at_ref[...], preferred_element_type=jnp.float32)
            c2 = jnp.dot(x1, bmt_ref[...], preferred_element_type=jnp.float32)
            pad = jnp.zeros_like(z)
            zt_ref[pl.ds(b * BN, BN), :] = jnp.concatenate([z, pad], axis=1)
            c2t_ref[pl.ds(b * BN, BN), :] = jnp.concatenate([c2, pad], axis=1)
            return carry

        lax.fori_loop(0, NBLK, body, 0)


def _final1_call(g3, bT, aT, bmT):
    return pl.pallas_call(
        _final1_body,
        grid=(NBLK,),
        in_specs=[
            pl.BlockSpec((BN, K, 2 * C), lambda i: (i, 0, 0)),
            pl.BlockSpec((BN, 2 * C), lambda i: (i, 0)),
            pl.BlockSpec((C, C), lambda i: (0, 0)),
            pl.BlockSpec((C, C), lambda i: (0, 0)),
        ],
        out_specs=[
            pl.BlockSpec((N, 2 * C), lambda i: (0, 0)),
            pl.BlockSpec((N, 2 * C), lambda i: (0, 0)),
        ],
        out_shape=[
            jax.ShapeDtypeStruct((N, 2 * C), jnp.float32),
            jax.ShapeDtypeStruct((N, 2 * C), jnp.float32),
        ],
        scratch_shapes=[pltpu.VMEM((N, K, 2 * C), jnp.float32),
                        pltpu.VMEM((2, 2 * C), jnp.float32)],
    )(g3, bT, aT, bmT)


def _final2_body(g_ref, bt_ref, out_ref, ybuf_ref, acc_ref):
    i = pl.program_id(0)
    _bn_step(g_ref, bt_ref, ybuf_ref, acc_ref, i)

    @pl.when(i == NBLK - 1)
    def _():
        mu, inv = _bn_inv(acc_ref)

        def body(b, carry):
            x2 = _bn_apply(ybuf_ref[pl.ds(b * BN, BN)], mu, inv)[:, 0:C]
            out_ref[:, pl.ds(b * BN, BN)] = jnp.transpose(x2)
            return carry

        lax.fori_loop(0, NBLK, body, 0)


def _final2_call(g3, bT):
    return pl.pallas_call(
        _final2_body,
        grid=(NBLK,),
        in_specs=[
            pl.BlockSpec((BN, K, 2 * C), lambda i: (i, 0, 0)),
            pl.BlockSpec((BN, 2 * C), lambda i: (i, 0)),
        ],
        out_specs=pl.BlockSpec((C, N), lambda i: (0, 0)),
        out_shape=jax.ShapeDtypeStruct((C, N), jnp.float32),
        scratch_shapes=[pltpu.VMEM((N, K, 2 * C), jnp.float32),
                        pltpu.VMEM((2, 2 * C), jnp.float32)],
    )(g3, bT)


def kernel(coordinates, features, W1, W2):
    x = coordinates[0]
    f = features[0]
    aT = jnp.transpose(W2[:, :C])
    bmT = jnp.transpose(W2[:, C:] - W2[:, :C])

    idx, uT, b1T = _knn_call(jnp.transpose(x), x, f, W1[:, C:], W1[:, :C])
    idx2d = idx.reshape(N * K // 128, 128)

    g1 = _sc_gather(uT, idx2d).reshape(N, K, 2 * C)
    zT, c2T = _final1_call(g1, b1T, aT, bmT)

    g2 = _sc_gather(zT, idx2d).reshape(N, K, 2 * C)
    x2 = _final2_call(g2, c2T)

    return x2[None]
